# baseline (speedup 1.0000x reference)
import functools

import jax
import jax.numpy as jnp
from jax import lax
from jax.experimental import pallas as pl
from jax.experimental.pallas import tpu as pltpu
from jax.experimental.pallas import tpu_sc as plsc

VOCAB = 33
HIDDEN = 1024
N_TAB = 6
CHUNK = 32
GRP = 16
UNROLL = 8


@jax.jit
def _gather6(idx2d, tabs):
    info = plsc.get_sparse_core_info()
    num_cores = info.num_cores
    nw = num_cores * info.num_subcores
    b = idx2d.shape[0] * idx2d.shape[1]
    b_per_w = b // nw
    n_chunks = b_per_w // CHUNK
    grp_per_w = b_per_w // GRP
    chunk_elems = CHUNK * HIDDEN
    w_elems = b_per_w * HIDDEN

    mesh = plsc.VectorSubcoreMesh(core_axis_name="c", subcore_axis_name="s")

    @functools.partial(
        pl.kernel,
        mesh=mesh,
        out_type=[jax.ShapeDtypeStruct((b * HIDDEN,), jnp.float32)] * N_TAB,
        scratch_types=[
            pltpu.VMEM((grp_per_w, GRP), jnp.int32),
            pltpu.VMEM((VOCAB * HIDDEN,), jnp.float32),
            pltpu.VMEM((chunk_elems,), jnp.float32),
            pltpu.VMEM((chunk_elems,), jnp.float32),
            pltpu.SemaphoreType.DMA,
            pltpu.SemaphoreType.DMA,
        ],
        compiler_params=pltpu.CompilerParams(needs_layout_passes=False),
    )
    def k(idx_hbm, tabs_hbm, out0, out1, out2, out3, out4, out5,
          idx_v, tab_v, buf0, buf1, sem0, sem1):
        outs = (out0, out1, out2, out3, out4, out5)
        bufs = (buf0, buf1)
        sems = (sem0, sem1)
        s_idx = lax.axis_index("s")
        c_idx = lax.axis_index("c")
        wid = s_idx * num_cores + c_idx
        base_e = wid * w_elems

        pltpu.sync_copy(idx_hbm.at[pl.ds(wid * grp_per_w, grp_per_w)],
                        idx_v)

        lane = lax.iota(jnp.int32, GRP)
        wrow = lane * HIDDEN

        def make_table(t):
            out_t = outs[t]

            def build(ck, bb):
                for g2 in range(CHUNK // GRP):
                    addr = idx_v[2 * ck + g2] * HIDDEN
                    wvec = wrow + g2 * (GRP * HIDDEN)

                    def h_body(h, carry, _addr=addr, _wvec=wvec, _bb=bb):
                        val = plsc.load_gather(tab_v, [_addr + h])
                        plsc.store_scatter(bufs[_bb], [_wvec + h], val)
                        return carry

                    lax.fori_loop(0, HIDDEN, h_body, 0, unroll=UNROLL)

            def wstart(ck, bb):
                pltpu.async_copy(
                    bufs[bb], out_t.at[pl.ds(base_e + ck * chunk_elems,
                                             chunk_elems)], sems[bb])

            def wwait(ck, bb):
                pltpu.make_async_copy(
                    bufs[bb], out_t.at[pl.ds(base_e + ck * chunk_elems,
                                             chunk_elems)], sems[bb]).wait()

            for bb in range(2):
                build(bb, bb)
                wstart(bb, bb)

            def pair(g, carry):
                for bb in range(2):
                    ck = 2 * g + bb
                    wwait(ck - 2, bb)
                    build(ck, bb)
                    wstart(ck, bb)
                return carry

            lax.fori_loop(1, n_chunks // 2, pair, 0)

            for bb in range(2):
                wwait(n_chunks - 2 + bb, bb)

        for t in range(N_TAB):
            pltpu.sync_copy(tabs_hbm.at[t], tab_v)
            make_table(t)

    return k(idx2d, tabs)


ROWS_BLK = 1024


def _tc_body(idx_ref, tabs_ref, *out_refs):
    idx = idx_ref[...]
    cols = jax.lax.broadcasted_iota(jnp.int32, (1, 64), 1)
    oh = (idx == cols).astype(jnp.float32)
    dn = (((1,), (0,)), ((), ()))
    for t in range(N_TAB):
        out_refs[t][...] = jax.lax.dot_general(
            oh, tabs_ref[t], dimension_numbers=dn,
            precision=jax.lax.Precision.HIGHEST,
            preferred_element_type=jnp.float32)


def _tc_gather6(idx_col, tabs_pad):
    b = idx_col.shape[0]
    grid = (b // ROWS_BLK,)
    return pl.pallas_call(
        _tc_body,
        grid=grid,
        in_specs=[
            pl.BlockSpec((ROWS_BLK, 1), lambda i: (i, 0)),
            pl.BlockSpec((N_TAB, 64, HIDDEN), lambda i: (0, 0, 0)),
        ],
        out_specs=[pl.BlockSpec((ROWS_BLK, HIDDEN), lambda i: (i, 0))
                   for _ in range(N_TAB)],
        out_shape=[jax.ShapeDtypeStruct((b, HIDDEN), jnp.float32)] * N_TAB,
    )(idx_col, tabs_pad)


def kernel(inputs, table_0, table_1, table_2, table_3, table_4, table_5):
    shape = inputs.shape
    idx_flat = inputs.reshape(-1).astype(jnp.int32)
    idx2d = idx_flat.reshape(-1, GRP)
    tabs = jnp.stack([table_0, table_1, table_2, table_3, table_4, table_5])
    tabs_flat = tabs.reshape(N_TAB, VOCAB * HIDDEN)
    outs = _gather6(idx2d, tabs_flat)
    tabs_pad = jnp.pad(tabs, ((0, 0), (0, 64 - VOCAB), (0, 0)))
    dups = _tc_gather6(idx_flat.reshape(-1, 1), tabs_pad)
    ve = [o.reshape(*shape, HIDDEN) for o in outs]
    dv = [o.reshape(*shape, HIDDEN) for o in dups]
    return tuple(ve + dv[::-1])


# speedup vs baseline: 9.1852x; 9.1852x over previous
import functools

import jax
import jax.numpy as jnp
from jax import lax
from jax.experimental import pallas as pl
from jax.experimental.pallas import tpu as pltpu
from jax.experimental.pallas import tpu_sc as plsc

VOCAB = 33
HIDDEN = 1024
N_TAB = 6
N_SC = 5
CHUNK = 32
TC_LIST = list(range(N_SC, N_TAB)) + list(range(N_TAB - 1, -1, -1))


@jax.jit
def _gather_sc(idx2d, tabs):
    info = plsc.get_sparse_core_info()
    num_cores = info.num_cores
    nw = num_cores * info.num_subcores
    b = idx2d.shape[0] * idx2d.shape[1]
    b_per_w = b // nw
    n_chunks = b_per_w // CHUNK

    mesh = plsc.VectorSubcoreMesh(core_axis_name="c", subcore_axis_name="s")

    @functools.partial(
        pl.kernel,
        mesh=mesh,
        out_type=(
            [jax.ShapeDtypeStruct((b, HIDDEN), jnp.float32)] * N_SC
            + [jax.ShapeDtypeStruct((nw, N_SC, VOCAB, HIDDEN), jnp.float32)]
        ),
        scratch_types=[
            pltpu.VMEM((n_chunks, CHUNK), jnp.int32),
            pltpu.VMEM((VOCAB, HIDDEN), jnp.float32),
            pltpu.VMEM((CHUNK, HIDDEN), jnp.float32),
            pltpu.VMEM((CHUNK, HIDDEN), jnp.float32),
            pltpu.SemaphoreType.DMA,
            pltpu.SemaphoreType.DMA,
        ],
    )
    def k(idx_hbm, tabs_hbm, *rest):
        outs = rest[:N_SC]
        reps = rest[N_SC]
        idx_v, tab_v, buf0, buf1, sem0, sem1 = rest[N_SC + 1:]
        bufs = (buf0, buf1)
        sems = (sem0, sem1)
        s_idx = lax.axis_index("s")
        c_idx = lax.axis_index("c")
        wid = s_idx * num_cores + c_idx
        base = wid * b_per_w

        for t in range(N_SC):
            pltpu.sync_copy(tabs_hbm.at[t], tab_v)
            pltpu.sync_copy(tab_v, reps.at[wid, t])

        pltpu.sync_copy(idx_hbm.at[pl.ds(wid * n_chunks, n_chunks)], idx_v)

        def gather_start(tab, ck, bb):
            pltpu.async_copy(tab.at[idx_v.at[ck]], bufs[bb], sems[bb])

        def gather_wait(tab, ck, bb):
            pltpu.make_async_copy(tab.at[idx_v.at[ck]], bufs[bb],
                                  sems[bb]).wait()

        for t in range(N_SC):
            tab = reps.at[wid, t]
            out = outs[t]

            for bb in range(2):
                gather_start(tab, bb, bb)

            def pair(g, carry, _tab=tab, _out=out):
                for bb in range(2):
                    ck = g * 2 + bb
                    gather_wait(_tab, ck, bb)
                    pltpu.sync_copy(bufs[bb],
                                    _out.at[pl.ds(base + ck * CHUNK, CHUNK)])
                    gather_start(_tab, ck + 2, bb)
                return carry

            lax.fori_loop(0, (n_chunks - 2) // 2, pair, 0)

            for bb in range(2):
                ck = n_chunks - 2 + bb
                gather_wait(tab, ck, bb)
                pltpu.sync_copy(bufs[bb],
                                out.at[pl.ds(base + ck * CHUNK, CHUNK)])

    return k(idx2d, tabs)[:N_SC]


ROWS_BLK = 512


def _tc_body(idx_ref, tabs_ref, *out_refs):
    idx = idx_ref[...]
    cols = jax.lax.broadcasted_iota(jnp.int32, (1, 64), 1)
    oh = (idx == cols).astype(jnp.float32)
    dn = (((1,), (0,)), ((), ()))
    res = {}
    for t in sorted(set(TC_LIST)):
        res[t] = jax.lax.dot_general(
            oh, tabs_ref[t], dimension_numbers=dn,
            preferred_element_type=jnp.float32)
    for j, t in enumerate(TC_LIST):
        out_refs[j][...] = res[t]


def _tc_gather(idx_col, tabs_pad):
    b = idx_col.shape[0]
    grid = (b // ROWS_BLK,)
    return pl.pallas_call(
        _tc_body,
        grid=grid,
        in_specs=[
            pl.BlockSpec((ROWS_BLK, 1), lambda i: (i, 0)),
            pl.BlockSpec((N_TAB, 64, HIDDEN), lambda i: (0, 0, 0)),
        ],
        out_specs=[pl.BlockSpec((ROWS_BLK, HIDDEN), lambda i: (i, 0))
                   for _ in TC_LIST],
        out_shape=[jax.ShapeDtypeStruct((b, HIDDEN), jnp.float32)
                   for _ in TC_LIST],
    )(idx_col, tabs_pad)


def kernel(inputs, table_0, table_1, table_2, table_3, table_4, table_5):
    shape = inputs.shape
    idx_flat = inputs.reshape(-1).astype(jnp.int32)
    idx2d = idx_flat.reshape(-1, CHUNK)
    tabs = jnp.stack([table_0, table_1, table_2, table_3, table_4, table_5])
    outs = _gather_sc(idx2d, tabs[:N_SC])
    tabs_pad = jnp.pad(tabs, ((0, 0), (0, 64 - VOCAB), (0, 0)))
    tc_outs = _tc_gather(idx_flat.reshape(-1, 1), tabs_pad)
    ve = [o.reshape(*shape, HIDDEN) for o in outs]
    tv = [o.reshape(*shape, HIDDEN) for o in tc_outs]
    return tuple(ve + tv)


# speedup vs baseline: 10.0078x; 1.0895x over previous
import functools

import jax
import jax.numpy as jnp
from jax import lax
from jax.experimental import pallas as pl
from jax.experimental.pallas import tpu as pltpu
from jax.experimental.pallas import tpu_sc as plsc

VOCAB = 33
HIDDEN = 1024
N_TAB = 6
N_SC = 4
CHUNK = 32
TC_LIST = list(range(N_SC, N_TAB)) + list(range(N_TAB - 1, -1, -1))


@jax.jit
def _gather_sc(idx2d, tabs):
    info = plsc.get_sparse_core_info()
    num_cores = info.num_cores
    nw = num_cores * info.num_subcores
    b = idx2d.shape[0] * idx2d.shape[1]
    b_per_w = b // nw
    n_chunks = b_per_w // CHUNK

    mesh = plsc.VectorSubcoreMesh(core_axis_name="c", subcore_axis_name="s")

    @functools.partial(
        pl.kernel,
        mesh=mesh,
        out_type=(
            [jax.ShapeDtypeStruct((b, HIDDEN), jnp.float32)] * N_SC
            + [jax.ShapeDtypeStruct((nw, N_SC, VOCAB, HIDDEN), jnp.float32)]
        ),
        scratch_types=[
            pltpu.VMEM((n_chunks, CHUNK), jnp.int32),
            pltpu.VMEM((VOCAB, HIDDEN), jnp.float32),
            pltpu.VMEM((CHUNK, HIDDEN), jnp.float32),
            pltpu.VMEM((CHUNK, HIDDEN), jnp.float32),
            pltpu.SemaphoreType.DMA,
            pltpu.SemaphoreType.DMA,
        ],
    )
    def k(idx_hbm, tabs_hbm, *rest):
        outs = rest[:N_SC]
        reps = rest[N_SC]
        idx_v, tab_v, buf0, buf1, sem0, sem1 = rest[N_SC + 1:]
        bufs = (buf0, buf1)
        sems = (sem0, sem1)
        s_idx = lax.axis_index("s")
        c_idx = lax.axis_index("c")
        wid = s_idx * num_cores + c_idx
        base = wid * b_per_w

        for t in range(N_SC):
            pltpu.sync_copy(tabs_hbm.at[t], tab_v)
            pltpu.sync_copy(tab_v, reps.at[wid, t])

        pltpu.sync_copy(idx_hbm.at[pl.ds(wid * n_chunks, n_chunks)], idx_v)

        def gather_start(tab, ck, bb):
            pltpu.async_copy(tab.at[idx_v.at[ck]], bufs[bb], sems[bb])

        def gather_wait(tab, ck, bb):
            pltpu.make_async_copy(tab.at[idx_v.at[ck]], bufs[bb],
                                  sems[bb]).wait()

        for t in range(N_SC):
            tab = reps.at[wid, t]
            out = outs[t]

            for bb in range(2):
                gather_start(tab, bb, bb)

            def pair(g, carry, _tab=tab, _out=out):
                for bb in range(2):
                    ck = g * 2 + bb
                    gather_wait(_tab, ck, bb)
                    pltpu.sync_copy(bufs[bb],
                                    _out.at[pl.ds(base + ck * CHUNK, CHUNK)])
                    gather_start(_tab, ck + 2, bb)
                return carry

            lax.fori_loop(0, (n_chunks - 2) // 2, pair, 0)

            for bb in range(2):
                ck = n_chunks - 2 + bb
                gather_wait(tab, ck, bb)
                pltpu.sync_copy(bufs[bb],
                                out.at[pl.ds(base + ck * CHUNK, CHUNK)])

    return k(idx2d, tabs)[:N_SC]


ROWS_BLK = 512


def _tc_body(idx_ref, tabs_ref, *out_refs):
    idx = idx_ref[...]
    cols = jax.lax.broadcasted_iota(jnp.int32, (1, 64), 1)
    oh = (idx == cols).astype(jnp.float32)
    dn = (((1,), (0,)), ((), ()))
    res = {}
    for t in sorted(set(TC_LIST)):
        res[t] = jax.lax.dot_general(
            oh, tabs_ref[t], dimension_numbers=dn,
            preferred_element_type=jnp.float32)
    for j, t in enumerate(TC_LIST):
        out_refs[j][...] = res[t]


def _tc_gather(idx_col, tabs_pad):
    b = idx_col.shape[0]
    grid = (b // ROWS_BLK,)
    return pl.pallas_call(
        _tc_body,
        grid=grid,
        in_specs=[
            pl.BlockSpec((ROWS_BLK, 1), lambda i: (i, 0)),
            pl.BlockSpec((N_TAB, 64, HIDDEN), lambda i: (0, 0, 0)),
        ],
        out_specs=[pl.BlockSpec((ROWS_BLK, HIDDEN), lambda i: (i, 0))
                   for _ in TC_LIST],
        out_shape=[jax.ShapeDtypeStruct((b, HIDDEN), jnp.float32)
                   for _ in TC_LIST],
    )(idx_col, tabs_pad)


def kernel(inputs, table_0, table_1, table_2, table_3, table_4, table_5):
    shape = inputs.shape
    idx_flat = inputs.reshape(-1).astype(jnp.int32)
    idx2d = idx_flat.reshape(-1, CHUNK)
    tabs = jnp.stack([table_0, table_1, table_2, table_3, table_4, table_5])
    outs = _gather_sc(idx2d, tabs[:N_SC])
    tabs_pad = jnp.pad(tabs, ((0, 0), (0, 64 - VOCAB), (0, 0)))
    tc_outs = _tc_gather(idx_flat.reshape(-1, 1), tabs_pad)
    ve = [o.reshape(*shape, HIDDEN) for o in outs]
    tv = [o.reshape(*shape, HIDDEN) for o in tc_outs]
    return tuple(ve + tv)


# speedup vs baseline: 10.9902x; 1.0982x over previous
import functools

import jax
import jax.numpy as jnp
from jax import lax
from jax.experimental import pallas as pl
from jax.experimental.pallas import tpu as pltpu
from jax.experimental.pallas import tpu_sc as plsc

VOCAB = 33
HIDDEN = 1024
N_TAB = 6
N_SC = 3
CHUNK = 32
TC_LIST = list(range(N_SC, N_TAB)) + list(range(N_TAB - 1, -1, -1))


@jax.jit
def _gather_sc(idx2d, tabs):
    info = plsc.get_sparse_core_info()
    num_cores = info.num_cores
    nw = num_cores * info.num_subcores
    b = idx2d.shape[0] * idx2d.shape[1]
    b_per_w = b // nw
    n_chunks = b_per_w // CHUNK

    mesh = plsc.VectorSubcoreMesh(core_axis_name="c", subcore_axis_name="s")

    @functools.partial(
        pl.kernel,
        mesh=mesh,
        out_type=(
            [jax.ShapeDtypeStruct((b, HIDDEN), jnp.float32)] * N_SC
            + [jax.ShapeDtypeStruct((nw, N_SC, VOCAB, HIDDEN), jnp.float32)]
        ),
        scratch_types=[
            pltpu.VMEM((n_chunks, CHUNK), jnp.int32),
            pltpu.VMEM((VOCAB, HIDDEN), jnp.float32),
            pltpu.VMEM((CHUNK, HIDDEN), jnp.float32),
            pltpu.VMEM((CHUNK, HIDDEN), jnp.float32),
            pltpu.SemaphoreType.DMA,
            pltpu.SemaphoreType.DMA,
        ],
    )
    def k(idx_hbm, tabs_hbm, *rest):
        outs = rest[:N_SC]
        reps = rest[N_SC]
        idx_v, tab_v, buf0, buf1, sem0, sem1 = rest[N_SC + 1:]
        bufs = (buf0, buf1)
        sems = (sem0, sem1)
        s_idx = lax.axis_index("s")
        c_idx = lax.axis_index("c")
        wid = s_idx * num_cores + c_idx
        base = wid * b_per_w

        for t in range(N_SC):
            pltpu.sync_copy(tabs_hbm.at[t], tab_v)
            pltpu.sync_copy(tab_v, reps.at[wid, t])

        pltpu.sync_copy(idx_hbm.at[pl.ds(wid * n_chunks, n_chunks)], idx_v)

        def gather_start(tab, ck, bb):
            pltpu.async_copy(tab.at[idx_v.at[ck]], bufs[bb], sems[bb])

        def gather_wait(tab, ck, bb):
            pltpu.make_async_copy(tab.at[idx_v.at[ck]], bufs[bb],
                                  sems[bb]).wait()

        for t in range(N_SC):
            tab = reps.at[wid, t]
            out = outs[t]

            for bb in range(2):
                gather_start(tab, bb, bb)

            def pair(g, carry, _tab=tab, _out=out):
                for bb in range(2):
                    ck = g * 2 + bb
                    gather_wait(_tab, ck, bb)
                    pltpu.sync_copy(bufs[bb],
                                    _out.at[pl.ds(base + ck * CHUNK, CHUNK)])
                    gather_start(_tab, ck + 2, bb)
                return carry

            lax.fori_loop(0, (n_chunks - 2) // 2, pair, 0)

            for bb in range(2):
                ck = n_chunks - 2 + bb
                gather_wait(tab, ck, bb)
                pltpu.sync_copy(bufs[bb],
                                out.at[pl.ds(base + ck * CHUNK, CHUNK)])

    return k(idx2d, tabs)[:N_SC]


ROWS_BLK = 512


def _tc_body(idx_ref, tabs_ref, *out_refs):
    idx = idx_ref[...]
    cols = jax.lax.broadcasted_iota(jnp.int32, (1, 64), 1)
    oh = (idx == cols).astype(jnp.float32)
    dn = (((1,), (0,)), ((), ()))
    res = {}
    for t in sorted(set(TC_LIST)):
        res[t] = jax.lax.dot_general(
            oh, tabs_ref[t], dimension_numbers=dn,
            preferred_element_type=jnp.float32)
    for j, t in enumerate(TC_LIST):
        out_refs[j][...] = res[t]


def _tc_gather(idx_col, tabs_pad):
    b = idx_col.shape[0]
    grid = (b // ROWS_BLK,)
    return pl.pallas_call(
        _tc_body,
        grid=grid,
        in_specs=[
            pl.BlockSpec((ROWS_BLK, 1), lambda i: (i, 0)),
            pl.BlockSpec((N_TAB, 64, HIDDEN), lambda i: (0, 0, 0)),
        ],
        out_specs=[pl.BlockSpec((ROWS_BLK, HIDDEN), lambda i: (i, 0))
                   for _ in TC_LIST],
        out_shape=[jax.ShapeDtypeStruct((b, HIDDEN), jnp.float32)
                   for _ in TC_LIST],
    )(idx_col, tabs_pad)


def kernel(inputs, table_0, table_1, table_2, table_3, table_4, table_5):
    shape = inputs.shape
    idx_flat = inputs.reshape(-1).astype(jnp.int32)
    idx2d = idx_flat.reshape(-1, CHUNK)
    tabs = jnp.stack([table_0, table_1, table_2, table_3, table_4, table_5])
    outs = _gather_sc(idx2d, tabs[:N_SC])
    tabs_pad = jnp.pad(tabs, ((0, 0), (0, 64 - VOCAB), (0, 0)))
    tc_outs = _tc_gather(idx_flat.reshape(-1, 1), tabs_pad)
    ve = [o.reshape(*shape, HIDDEN) for o in outs]
    tv = [o.reshape(*shape, HIDDEN) for o in tc_outs]
    return tuple(ve + tv)


# speedup vs baseline: 11.6649x; 1.0614x over previous
import functools

import jax
import jax.numpy as jnp
from jax import lax
from jax.experimental import pallas as pl
from jax.experimental.pallas import tpu as pltpu
from jax.experimental.pallas import tpu_sc as plsc

VOCAB = 33
HIDDEN = 1024
N_TAB = 6
N_SC = 2
CHUNK = 32
TC_LIST = list(range(N_SC, N_TAB)) + list(range(N_TAB - 1, N_SC - 1, -1))


@jax.jit
def _gather_sc(idx2d, tabs):
    info = plsc.get_sparse_core_info()
    num_cores = info.num_cores
    nw = num_cores * info.num_subcores
    b = idx2d.shape[0] * idx2d.shape[1]
    b_per_w = b // nw
    n_chunks = b_per_w // CHUNK

    mesh = plsc.VectorSubcoreMesh(core_axis_name="c", subcore_axis_name="s")

    @functools.partial(
        pl.kernel,
        mesh=mesh,
        out_type=(
            [jax.ShapeDtypeStruct((b, HIDDEN), jnp.float32)] * (2 * N_SC)
            + [jax.ShapeDtypeStruct((nw, N_SC, VOCAB, HIDDEN), jnp.float32)]
        ),
        scratch_types=[
            pltpu.VMEM((n_chunks, CHUNK), jnp.int32),
            pltpu.VMEM((VOCAB, HIDDEN), jnp.float32),
            pltpu.VMEM((CHUNK, HIDDEN), jnp.float32),
            pltpu.VMEM((CHUNK, HIDDEN), jnp.float32),
            pltpu.SemaphoreType.DMA,
            pltpu.SemaphoreType.DMA,
        ],
    )
    def k(idx_hbm, tabs_hbm, *rest):
        outs = rest[:N_SC]
        douts = rest[N_SC:2 * N_SC]
        reps = rest[2 * N_SC]
        idx_v, tab_v, buf0, buf1, sem0, sem1 = rest[2 * N_SC + 1:]
        bufs = (buf0, buf1)
        sems = (sem0, sem1)
        s_idx = lax.axis_index("s")
        c_idx = lax.axis_index("c")
        wid = s_idx * num_cores + c_idx
        base = wid * b_per_w

        for t in range(N_SC):
            pltpu.sync_copy(tabs_hbm.at[t], tab_v)
            pltpu.sync_copy(tab_v, reps.at[wid, t])

        pltpu.sync_copy(idx_hbm.at[pl.ds(wid * n_chunks, n_chunks)], idx_v)

        def gather_start(tab, ck, bb):
            pltpu.async_copy(tab.at[idx_v.at[ck]], bufs[bb], sems[bb])

        def gather_wait(tab, ck, bb):
            pltpu.make_async_copy(tab.at[idx_v.at[ck]], bufs[bb],
                                  sems[bb]).wait()

        for t in range(N_SC):
            tab = reps.at[wid, t]
            out = outs[t]
            dout = douts[t]

            for bb in range(2):
                gather_start(tab, bb, bb)

            def pair(g, carry, _tab=tab, _out=out, _dout=dout):
                for bb in range(2):
                    ck = g * 2 + bb
                    gather_wait(_tab, ck, bb)
                    sl = pl.ds(base + ck * CHUNK, CHUNK)
                    pltpu.sync_copy(bufs[bb], _out.at[sl])
                    pltpu.sync_copy(bufs[bb], _dout.at[sl])
                    gather_start(_tab, ck + 2, bb)
                return carry

            lax.fori_loop(0, (n_chunks - 2) // 2, pair, 0)

            for bb in range(2):
                ck = n_chunks - 2 + bb
                gather_wait(tab, ck, bb)
                sl = pl.ds(base + ck * CHUNK, CHUNK)
                pltpu.sync_copy(bufs[bb], out.at[sl])
                pltpu.sync_copy(bufs[bb], dout.at[sl])

    return k(idx2d, tabs)[:2 * N_SC]


ROWS_BLK = 512


def _tc_body(idx_ref, tabs_ref, *out_refs):
    idx = idx_ref[...]
    cols = jax.lax.broadcasted_iota(jnp.int32, (1, 64), 1)
    oh = (idx == cols).astype(jnp.float32)
    dn = (((1,), (0,)), ((), ()))
    res = {}
    for t in sorted(set(TC_LIST)):
        res[t] = jax.lax.dot_general(
            oh, tabs_ref[t], dimension_numbers=dn,
            preferred_element_type=jnp.float32)
    for j, t in enumerate(TC_LIST):
        out_refs[j][...] = res[t]


def _tc_gather(idx_col, tabs_pad):
    b = idx_col.shape[0]
    grid = (b // ROWS_BLK,)
    return pl.pallas_call(
        _tc_body,
        grid=grid,
        in_specs=[
            pl.BlockSpec((ROWS_BLK, 1), lambda i: (i, 0)),
            pl.BlockSpec((N_TAB, 64, HIDDEN), lambda i: (0, 0, 0)),
        ],
        out_specs=[pl.BlockSpec((ROWS_BLK, HIDDEN), lambda i: (i, 0))
                   for _ in TC_LIST],
        out_shape=[jax.ShapeDtypeStruct((b, HIDDEN), jnp.float32)
                   for _ in TC_LIST],
    )(idx_col, tabs_pad)


def kernel(inputs, table_0, table_1, table_2, table_3, table_4, table_5):
    shape = inputs.shape
    idx_flat = inputs.reshape(-1).astype(jnp.int32)
    idx2d = idx_flat.reshape(-1, CHUNK)
    tabs = jnp.stack([table_0, table_1, table_2, table_3, table_4, table_5])
    sc_outs = _gather_sc(idx2d, tabs[:N_SC])
    tabs_pad = jnp.pad(tabs, ((0, 0), (0, 64 - VOCAB), (0, 0)))
    tc_outs = _tc_gather(idx_flat.reshape(-1, 1), tabs_pad)
    sc_u = [o.reshape(*shape, HIDDEN) for o in sc_outs[:N_SC]]
    sc_d = [o.reshape(*shape, HIDDEN) for o in sc_outs[N_SC:]]
    tv = [o.reshape(*shape, HIDDEN) for o in tc_outs]
    return tuple(sc_u + tv[:N_TAB - N_SC]
                 + tv[N_TAB - N_SC:] + sc_d[::-1])
